# baseline (speedup 1.0000x reference)
import functools

import jax
import jax.numpy as jnp
from jax import lax
from jax.experimental import pallas as pl
from jax.experimental.pallas import tpu as pltpu
from jax.experimental.pallas import tpu_sc as plsc

_D = 64
_ROW = 2 * _D
_EG = 0.57721566490153286
_EPS = 1e-23
_NC, _NS = 2, 16
_NW = _NC * _NS
_GCHUNK = 128


def _sc_gather(gci_flat, table):
    rows_total = gci_flat.shape[0]
    r_per_w = rows_total // _NW
    n_chunks = r_per_w // _GCHUNK
    mesh = plsc.VectorSubcoreMesh(core_axis_name="c", subcore_axis_name="s")

    @functools.partial(
        pl.kernel,
        out_type=jax.ShapeDtypeStruct((rows_total, _ROW), jnp.float32),
        mesh=mesh,
        scratch_types=[
            pltpu.VMEM((r_per_w,), jnp.int32),
            pltpu.VMEM((_GCHUNK, _ROW), jnp.float32),
            pltpu.VMEM((_GCHUNK, _ROW), jnp.float32),
            pltpu.SemaphoreType.DMA,
            pltpu.SemaphoreType.DMA,
        ],
    )
    def gather_kernel(gci_hbm, table_hbm, out_hbm, idx_v, rows_a, rows_b, sem_a, sem_b):
        wid = lax.axis_index("s") * _NC + lax.axis_index("c")
        base = wid * r_per_w
        pltpu.sync_copy(gci_hbm.at[pl.ds(base, r_per_w)], idx_v)
        bufs = ((rows_a, sem_a), (rows_b, sem_b))
        pltpu.async_copy(
            table_hbm.at[idx_v.at[pl.ds(0, _GCHUNK)]], rows_a, sem_a)
        for j in range(n_chunks):
            buf, sem = bufs[j % 2]
            nbuf, nsem = bufs[(j + 1) % 2]
            if j + 1 < n_chunks:
                pltpu.async_copy(
                    table_hbm.at[idx_v.at[pl.ds((j + 1) * _GCHUNK, _GCHUNK)]],
                    nbuf, nsem)
            pltpu.make_async_copy(
                table_hbm.at[idx_v.at[pl.ds(j * _GCHUNK, _GCHUNK)]], buf, sem
            ).wait()
            pltpu.sync_copy(buf, out_hbm.at[pl.ds(base + j * _GCHUNK, _GCHUNK)])

    return gather_kernel(gci_flat, table)


def _softplus(x):
    return jnp.logaddexp(x, 0.0)


def _tc_compute(pairs):
    batch = pairs.shape[0]
    blk = 2048
    grid = batch // blk

    def body(x_ref, o_ref):
        x = x_ref[...]
        z_sub = x[:, 0 * _D:1 * _D]
        d_sub = x[:, 1 * _D:2 * _D]
        z_sup = x[:, 2 * _D:3 * _D]
        d_sup = x[:, 3 * _D:4 * _D]
        sp_sub = _softplus(d_sub)
        Z_sub = z_sub + sp_sub
        Z_sup = z_sup + _softplus(d_sup)
        z_meet = jnp.logaddexp(z_sub, z_sup)
        z_meet = jnp.maximum(z_meet, jnp.maximum(z_sub, z_sup))
        Z_meet = -jnp.logaddexp(-Z_sub, -Z_sup)
        Z_meet = jnp.minimum(Z_meet, jnp.minimum(Z_sub, Z_sup))
        c = 2.0 * _EG
        lv_meet = jnp.sum(
            jnp.log(_softplus(Z_meet - z_meet - c) + _EPS), axis=-1)
        lv_sub = jnp.sum(jnp.log(_softplus(sp_sub - c) + _EPS), axis=-1)
        o_ref[...] = jnp.clip(jnp.exp(lv_meet - lv_sub), 0.0, 1.0)

    return pl.pallas_call(
        body,
        grid=(grid,),
        in_specs=[pl.BlockSpec((blk, 4 * _D), lambda i: (i, 0))],
        out_specs=pl.BlockSpec((blk,), lambda i: (i,)),
        out_shape=jax.ShapeDtypeStruct((batch,), jnp.float32),
    )(pairs)


def kernel(gci, table):
    batch = gci.shape[0]
    gathered = _sc_gather(gci.reshape(-1), table)
    return _tc_compute(gathered.reshape(batch, 2 * _ROW))


# speedup vs baseline: 1.2623x; 1.2623x over previous
import functools

import jax
import jax.numpy as jnp
import numpy as np
from jax import lax
from jax.experimental import pallas as pl
from jax.experimental.pallas import tpu as pltpu
from jax.experimental.pallas import tpu_sc as plsc

_D = 64
_ROW = 2 * _D
_EG = 0.57721566490153286
_EPS = 1e-23
_NC, _NS = 2, 16
_NW = _NC * _NS
_GCHUNK = 128


def _sc_gather(gci_flat, table):
    rows_total = gci_flat.shape[0]
    r_per_w = rows_total // _NW
    n_chunks = r_per_w // _GCHUNK
    mesh = plsc.VectorSubcoreMesh(core_axis_name="c", subcore_axis_name="s")

    @functools.partial(
        pl.kernel,
        out_type=jax.ShapeDtypeStruct((rows_total, _ROW), jnp.float32),
        mesh=mesh,
        scratch_types=[
            pltpu.VMEM((r_per_w,), jnp.int32),
            pltpu.VMEM((_GCHUNK, _ROW), jnp.float32),
            pltpu.VMEM((_GCHUNK, _ROW), jnp.float32),
            pltpu.SemaphoreType.DMA,
            pltpu.SemaphoreType.DMA,
        ],
    )
    def gather_kernel(gci_hbm, table_hbm, out_hbm, idx_v, rows_a, rows_b, sem_a, sem_b):
        wid = lax.axis_index("s") * _NC + lax.axis_index("c")
        base = wid * r_per_w
        pltpu.sync_copy(gci_hbm.at[pl.ds(base, r_per_w)], idx_v)
        bufs = ((rows_a, sem_a), (rows_b, sem_b))
        pltpu.async_copy(
            table_hbm.at[idx_v.at[pl.ds(0, _GCHUNK)]], rows_a, sem_a)
        for j in range(n_chunks):
            buf, sem = bufs[j % 2]
            nbuf, nsem = bufs[(j + 1) % 2]
            if j + 1 < n_chunks:
                pltpu.async_copy(
                    table_hbm.at[idx_v.at[pl.ds((j + 1) * _GCHUNK, _GCHUNK)]],
                    nbuf, nsem)
            pltpu.make_async_copy(
                table_hbm.at[idx_v.at[pl.ds(j * _GCHUNK, _GCHUNK)]], buf, sem
            ).wait()
            pltpu.sync_copy(buf, out_hbm.at[pl.ds(base + j * _GCHUNK, _GCHUNK)])

    return gather_kernel(gci_flat, table)


def _tc_compute(pairs):
    batch = pairs.shape[0]
    blk = 2048
    grid = batch // blk
    inv_k = float(np.exp(-2.0 * _EG))

    def body(x_ref, o_ref):
        x = x_ref[...]
        z_sub = x[:, 0 * _D:1 * _D]
        d_sub = x[:, 1 * _D:2 * _D]
        z_sup = x[:, 2 * _D:3 * _D]
        d_sup = x[:, 3 * _D:4 * _D]
        ea = jnp.exp(z_sub)
        eb = jnp.exp(z_sup)
        pda = 1.0 + jnp.exp(d_sub)
        pdb = 1.0 + jnp.exp(d_sup)
        big_a = ea * pda
        big_b = eb * pdb
        s = ea + eb
        t = big_a + big_b
        pm = big_a * big_b
        num = jnp.log1p(pm / (t * s) * inv_k) + _EPS
        den = jnp.log1p(pda * inv_k) + _EPS
        lsum = jnp.sum(jnp.log(num / den), axis=-1)
        o_ref[...] = jnp.clip(jnp.exp(lsum), 0.0, 1.0)

    return pl.pallas_call(
        body,
        grid=(grid,),
        in_specs=[pl.BlockSpec((blk, 4 * _D), lambda i: (i, 0))],
        out_specs=pl.BlockSpec((blk,), lambda i: (i,)),
        out_shape=jax.ShapeDtypeStruct((batch,), jnp.float32),
    )(pairs)


def kernel(gci, table):
    batch = gci.shape[0]
    gathered = _sc_gather(gci.reshape(-1), table)
    return _tc_compute(gathered.reshape(batch, 2 * _ROW))
